# baseline (speedup 1.0000x reference)
import functools

import jax
import jax.numpy as jnp
from jax import lax
from jax.experimental import pallas as pl
from jax.experimental.pallas import tpu as pltpu
from jax.experimental.pallas import tpu_sc as plsc

N_ANCHORS = 20000
BATCH = 8
M_GT = 100
N_CLASSES = 20
POS_IOU_THR = 0.3

NC, NS, L = 2, 16, 16
NW = NC * NS
NAP = 20480
TN = 640
MP = 112
NV = TN // L
GROUP = 4
NGROUPS = 10

_LN2 = 0.6931471805599453
_SQRT2 = 1.4142135623730951


def _vlog(x):
    bits = lax.bitcast_convert_type(x, jnp.int32)
    e = lax.shift_right_logical(bits, 23) - 127
    m = lax.bitcast_convert_type(
        (bits & 0x7FFFFF) | 0x3F800000, jnp.float32)
    big = m > _SQRT2
    m = jnp.where(big, m * 0.5, m)
    e = jnp.where(big, e + 1, e)
    z = (m - 1.0) / (m + 1.0)
    z2 = z * z
    p = ((z2 * (1.0 / 9.0) + (1.0 / 7.0)) * z2 + (1.0 / 5.0)) * z2 + (1.0 / 3.0)
    logm = 2.0 * z * (p * z2 + 1.0)
    return e.astype(jnp.float32) * _LN2 + logm


def _sc_body(anc_hbm, gt_hbm, pos_hbm, cls_hbm, reg_hbm,
             anc_raw, anc_der, gt_raw, gt_der, pos_buf, cls_buf, reg_buf):
    wid = lax.axis_index("s") * NC + lax.axis_index("c")
    base = wid * TN

    pltpu.sync_copy(anc_hbm.at[:, pl.ds(base, TN)], anc_raw)
    pltpu.sync_copy(gt_hbm, gt_raw)

    iota = lax.iota(jnp.int32, L)
    iota20 = iota * N_CLASSES
    iota5 = iota * 5
    zeros = jnp.zeros((L,), jnp.float32)
    ones = jnp.ones((L,), jnp.float32)

    def anc_chunk(i, _):
        o = i * L
        x1 = anc_raw[0, pl.ds(o, L)]
        y1 = anc_raw[1, pl.ds(o, L)]
        x2 = anc_raw[2, pl.ds(o, L)]
        y2 = anc_raw[3, pl.ds(o, L)]
        dx = x2 - x1
        dy = y2 - y1
        anc_der[0, pl.ds(o, L)] = dx * dy
        ex_w = jnp.maximum(dx, 1.0)
        ex_h = jnp.maximum(dy, 1.0)
        cx = x1 + 0.5 * ex_w
        cy = y1 + 0.5 * ex_h
        anc_der[1, pl.ds(o, L)] = cx.astype(jnp.int32).astype(jnp.float32)
        anc_der[2, pl.ds(o, L)] = cy.astype(jnp.int32).astype(jnp.float32)
        anc_der[3, pl.ds(o, L)] = _vlog(ex_w)
        anc_der[4, pl.ds(o, L)] = _vlog(ex_h)
        return 0

    lax.fori_loop(0, NV, anc_chunk, 0)

    def gt_batch(b, _):
        gbase = b * 6 * MP

        def gt_chunk(p, _):
            o = p * L
            x1 = gt_raw[pl.ds(gbase + o, L)]
            y1 = gt_raw[pl.ds(gbase + MP + o, L)]
            x2 = gt_raw[pl.ds(gbase + 2 * MP + o, L)]
            y2 = gt_raw[pl.ds(gbase + 3 * MP + o, L)]
            clsf = gt_raw[pl.ds(gbase + 4 * MP + o, L)]
            dx = x2 - x1
            dy = y2 - y1
            gt_der[pl.ds(gbase + o, L)] = dx * dy
            gw = jnp.maximum(dx, 1.0)
            gh = jnp.maximum(dy, 1.0)
            gt_der[pl.ds(gbase + MP + o, L)] = x1 + 0.5 * gw
            gt_der[pl.ds(gbase + 2 * MP + o, L)] = y1 + 0.5 * gh
            gt_der[pl.ds(gbase + 3 * MP + o, L)] = _vlog(gw)
            gt_der[pl.ds(gbase + 4 * MP + o, L)] = _vlog(gh)
            gt_der[pl.ds(gbase + 5 * MP + o, L)] = clsf
            return 0

        lax.fori_loop(0, MP // L, gt_chunk, 0)
        return 0

    lax.fori_loop(0, BATCH, gt_batch, 0)

    def make_group(nv):
        def run(b, gb):
            gbase = b * 6 * MP
            ax1 = [anc_raw[0, pl.ds(gb + v * L, L)] for v in range(nv)]
            ay1 = [anc_raw[1, pl.ds(gb + v * L, L)] for v in range(nv)]
            ax2 = [anc_raw[2, pl.ds(gb + v * L, L)] for v in range(nv)]
            ay2 = [anc_raw[3, pl.ds(gb + v * L, L)] for v in range(nv)]
            aar = [anc_der[0, pl.ds(gb + v * L, L)] for v in range(nv)]

            def jbody(j, carry):
                bests, bestis = carry
                jv = jnp.full((L,), j, jnp.int32)
                idx0 = jv + gbase
                gx1 = plsc.load_gather(gt_raw, [idx0])
                gy1 = plsc.load_gather(gt_raw, [idx0 + MP])
                gx2 = plsc.load_gather(gt_raw, [idx0 + 2 * MP])
                gy2 = plsc.load_gather(gt_raw, [idx0 + 3 * MP])
                ab = plsc.load_gather(gt_der, [idx0])
                nb, ni = [], []
                for v in range(nv):
                    ltx = jnp.maximum(ax1[v], gx1)
                    lty = jnp.maximum(ay1[v], gy1)
                    rbx = jnp.minimum(ax2[v], gx2)
                    rby = jnp.minimum(ay2[v], gy2)
                    wx = jnp.maximum(rbx - ltx, 0.0)
                    wy = jnp.maximum(rby - lty, 0.0)
                    inter = wx * wy
                    den = (aar[v] + ab) - inter + 1e-10
                    iou = inter / den
                    better = iou > bests[v]
                    nb.append(jnp.where(better, iou, bests[v]))
                    ni.append(jnp.where(better, jv, bestis[v]))
                return tuple(nb), tuple(ni)

            init = (tuple(jnp.full((L,), -1.0, jnp.float32) for _ in range(nv)),
                    tuple(jnp.zeros((L,), jnp.int32) for _ in range(nv)))
            bests, bestis = lax.fori_loop(0, M_GT, jbody, init)

            for v in range(nv):
                o = gb + v * L
                pos = bests[v] >= POS_IOU_THR
                pos_buf[pl.ds(o, L)] = jnp.where(pos, ones, zeros)
                bi = bestis[v] + gbase
                gcx = plsc.load_gather(gt_der, [bi + MP])
                gcy = plsc.load_gather(gt_der, [bi + 2 * MP])
                lgw = plsc.load_gather(gt_der, [bi + 3 * MP])
                lgh = plsc.load_gather(gt_der, [bi + 4 * MP])
                clsf = plsc.load_gather(gt_der, [bi + 5 * MP])
                tcx = anc_der[1, pl.ds(o, L)]
                tcy = anc_der[2, pl.ds(o, L)]
                lw = anc_der[3, pl.ds(o, L)]
                lh = anc_der[4, pl.ds(o, L)]
                ridx = iota5 + o * 5
                plsc.store_scatter(reg_buf, [ridx],
                                   jnp.where(pos, gcx - tcx, 0.0))
                plsc.store_scatter(reg_buf, [ridx + 1],
                                   jnp.where(pos, gcy - tcy, 0.0))
                plsc.store_scatter(reg_buf, [ridx + 2],
                                   jnp.where(pos, lgw - lw, 0.0))
                plsc.store_scatter(reg_buf, [ridx + 3],
                                   jnp.where(pos, lgh - lh, 0.0))
                plsc.store_scatter(reg_buf, [ridx + 4], zeros)
                cidx = (clsf.astype(jnp.int32) + iota20) + o * N_CLASSES
                plsc.store_scatter(cls_buf, [cidx], ones, mask=pos)

        return run

    group4 = make_group(GROUP)

    def batch_body(b, _):
        def zero_chunk(i, _):
            o = i * (20 * L)
            for u in range(20):
                cls_buf[pl.ds(o + u * L, L)] = zeros
            return 0

        lax.fori_loop(0, NV, zero_chunk, 0)

        def group_body(g, _):
            group4(b, g * (GROUP * L))
            return 0

        lax.fori_loop(0, NGROUPS, group_body, 0)

        pltpu.sync_copy(pos_buf, pos_hbm.at[b, pl.ds(base, TN)])
        pltpu.sync_copy(
            cls_buf, cls_hbm.at[b, pl.ds(base * N_CLASSES, TN * N_CLASSES)])
        pltpu.sync_copy(reg_buf, reg_hbm.at[b, pl.ds(base * 5, TN * 5)])
        return 0

    lax.fori_loop(0, BATCH, batch_body, 0)


@functools.partial(
    pl.kernel,
    out_type=(
        jax.ShapeDtypeStruct((BATCH, NAP), jnp.float32),
        jax.ShapeDtypeStruct((BATCH, NAP * N_CLASSES), jnp.float32),
        jax.ShapeDtypeStruct((BATCH, NAP * 5), jnp.float32),
    ),
    mesh=plsc.VectorSubcoreMesh(
        core_axis_name="c", subcore_axis_name="s",
        num_cores=NC, num_subcores=NS),
    compiler_params=pltpu.CompilerParams(needs_layout_passes=False),
    scratch_types=(
        pltpu.VMEM((4, TN), jnp.float32),
        pltpu.VMEM((5, TN), jnp.float32),
        pltpu.VMEM((BATCH * 6 * MP,), jnp.float32),
        pltpu.VMEM((BATCH * 6 * MP,), jnp.float32),
        pltpu.VMEM((TN,), jnp.float32),
        pltpu.VMEM((TN * N_CLASSES,), jnp.float32),
        pltpu.VMEM((TN * 5,), jnp.float32),
    ),
)
def _label_assign_sc(anc_hbm, gt_hbm, pos_hbm, cls_hbm, reg_hbm,
                     anc_raw, anc_der, gt_raw, gt_der,
                     pos_buf, cls_buf, reg_buf):
    _sc_body(anc_hbm, gt_hbm, pos_hbm, cls_hbm, reg_hbm,
             anc_raw, anc_der, gt_raw, gt_der, pos_buf, cls_buf, reg_buf)


def kernel(anchor, target, regressions, classifications):
    del regressions, classifications
    anc_t = jnp.zeros((4, NAP), jnp.float32).at[:, :N_ANCHORS].set(anchor.T)
    pad_row = jnp.array([0.0, 0.0, 1.0, 1.0, 0.0, 0.0], jnp.float32)
    tgt = jnp.concatenate(
        [target, jnp.broadcast_to(pad_row, (BATCH, MP - M_GT, 6))], axis=1)
    gt_t = jnp.transpose(tgt, (0, 2, 1)).reshape(-1)

    pos, cls, reg = _label_assign_sc(anc_t, gt_t)

    positive = pos[:, :N_ANCHORS] > 0.5
    cls_out = cls.reshape(BATCH, NAP, N_CLASSES)[:, :N_ANCHORS]
    reg_out = reg.reshape(BATCH, NAP, 5)[:, :N_ANCHORS]
    return positive, cls_out, reg_out


# speedup vs baseline: 1.0233x; 1.0233x over previous
import functools

import jax
import jax.numpy as jnp
from jax import lax
from jax.experimental import pallas as pl
from jax.experimental.pallas import tpu as pltpu
from jax.experimental.pallas import tpu_sc as plsc

N_ANCHORS = 20000
BATCH = 8
M_GT = 100
N_CLASSES = 20
POS_IOU_THR = 0.3

NC, NS, L = 2, 16, 16
NW = NC * NS
NAP = 20480
TN = 640
MP = 112
NV = TN // L
GROUP = 4
NGROUPS = 10

_LN2 = 0.6931471805599453
_SQRT2 = 1.4142135623730951


def _vlog(x):
    bits = lax.bitcast_convert_type(x, jnp.int32)
    e = lax.shift_right_logical(bits, 23) - 127
    m = lax.bitcast_convert_type(
        (bits & 0x7FFFFF) | 0x3F800000, jnp.float32)
    big = m > _SQRT2
    m = jnp.where(big, m * 0.5, m)
    e = jnp.where(big, e + 1, e)
    z = (m - 1.0) / (m + 1.0)
    z2 = z * z
    p = ((z2 * (1.0 / 9.0) + (1.0 / 7.0)) * z2 + (1.0 / 5.0)) * z2 + (1.0 / 3.0)
    logm = 2.0 * z * (p * z2 + 1.0)
    return e.astype(jnp.float32) * _LN2 + logm


def _sc_body(anc_hbm, gt_hbm, pos_hbm, cls_hbm, reg_hbm,
             anc_raw, anc_der, gt_raw, gt_der, pos_buf, cls_buf, reg_buf):
    wid = lax.axis_index("s") * NC + lax.axis_index("c")
    base = wid * TN

    pltpu.sync_copy(anc_hbm.at[:, pl.ds(base, TN)], anc_raw)
    pltpu.sync_copy(gt_hbm, gt_raw)

    iota = lax.iota(jnp.int32, L)
    iota20 = iota * N_CLASSES
    iota5 = iota * 5
    zeros = jnp.zeros((L,), jnp.float32)
    ones = jnp.ones((L,), jnp.float32)

    def anc_chunk(i, _):
        o = i * L
        x1 = anc_raw[0, pl.ds(o, L)]
        y1 = anc_raw[1, pl.ds(o, L)]
        x2 = anc_raw[2, pl.ds(o, L)]
        y2 = anc_raw[3, pl.ds(o, L)]
        dx = x2 - x1
        dy = y2 - y1
        anc_der[0, pl.ds(o, L)] = dx * dy
        ex_w = jnp.maximum(dx, 1.0)
        ex_h = jnp.maximum(dy, 1.0)
        cx = x1 + 0.5 * ex_w
        cy = y1 + 0.5 * ex_h
        anc_der[1, pl.ds(o, L)] = cx.astype(jnp.int32).astype(jnp.float32)
        anc_der[2, pl.ds(o, L)] = cy.astype(jnp.int32).astype(jnp.float32)
        anc_der[3, pl.ds(o, L)] = _vlog(ex_w)
        anc_der[4, pl.ds(o, L)] = _vlog(ex_h)
        return 0

    lax.fori_loop(0, NV, anc_chunk, 0)

    def gt_batch(b, _):
        gbase = b * 6 * MP

        def gt_chunk(p, _):
            o = p * L
            x1 = gt_raw[pl.ds(gbase + o, L)]
            y1 = gt_raw[pl.ds(gbase + MP + o, L)]
            x2 = gt_raw[pl.ds(gbase + 2 * MP + o, L)]
            y2 = gt_raw[pl.ds(gbase + 3 * MP + o, L)]
            clsf = gt_raw[pl.ds(gbase + 4 * MP + o, L)]
            dx = x2 - x1
            dy = y2 - y1
            gt_der[pl.ds(gbase + o, L)] = dx * dy
            gw = jnp.maximum(dx, 1.0)
            gh = jnp.maximum(dy, 1.0)
            gt_der[pl.ds(gbase + MP + o, L)] = x1 + 0.5 * gw
            gt_der[pl.ds(gbase + 2 * MP + o, L)] = y1 + 0.5 * gh
            gt_der[pl.ds(gbase + 3 * MP + o, L)] = _vlog(gw)
            gt_der[pl.ds(gbase + 4 * MP + o, L)] = _vlog(gh)
            gt_der[pl.ds(gbase + 5 * MP + o, L)] = clsf
            return 0

        lax.fori_loop(0, MP // L, gt_chunk, 0)
        return 0

    lax.fori_loop(0, BATCH, gt_batch, 0)

    def make_group(nv):
        def run(b, gb):
            gbase = b * 6 * MP
            ax1 = [anc_raw[0, pl.ds(gb + v * L, L)] for v in range(nv)]
            ay1 = [anc_raw[1, pl.ds(gb + v * L, L)] for v in range(nv)]
            ax2 = [anc_raw[2, pl.ds(gb + v * L, L)] for v in range(nv)]
            ay2 = [anc_raw[3, pl.ds(gb + v * L, L)] for v in range(nv)]
            aar = [anc_der[0, pl.ds(gb + v * L, L)] for v in range(nv)]

            def jbody(j, carry):
                bests, bestis = carry
                jv = jnp.full((L,), j, jnp.int32)
                idx0 = jv + gbase
                gx1 = plsc.load_gather(gt_raw, [idx0])
                gy1 = plsc.load_gather(gt_raw, [idx0 + MP])
                gx2 = plsc.load_gather(gt_raw, [idx0 + 2 * MP])
                gy2 = plsc.load_gather(gt_raw, [idx0 + 3 * MP])
                ab = plsc.load_gather(gt_der, [idx0])
                nb, ni = [], []
                for v in range(nv):
                    ltx = jnp.maximum(ax1[v], gx1)
                    lty = jnp.maximum(ay1[v], gy1)
                    rbx = jnp.minimum(ax2[v], gx2)
                    rby = jnp.minimum(ay2[v], gy2)
                    wx = jnp.maximum(rbx - ltx, 0.0)
                    wy = jnp.maximum(rby - lty, 0.0)
                    inter = wx * wy
                    den = (aar[v] + ab) - inter + 1e-10
                    iou = inter / den
                    better = iou > bests[v]
                    nb.append(jnp.where(better, iou, bests[v]))
                    ni.append(jnp.where(better, jv, bestis[v]))
                return tuple(nb), tuple(ni)

            init = (tuple(jnp.full((L,), -1.0, jnp.float32) for _ in range(nv)),
                    tuple(jnp.zeros((L,), jnp.int32) for _ in range(nv)))
            bests, bestis = lax.fori_loop(0, M_GT, jbody, init)

            for v in range(nv):
                o = gb + v * L
                pos = bests[v] >= POS_IOU_THR
                pos_buf[pl.ds(o, L)] = jnp.where(pos, ones, zeros)
                bi = bestis[v] + gbase
                gcx = plsc.load_gather(gt_der, [bi + MP])
                gcy = plsc.load_gather(gt_der, [bi + 2 * MP])
                lgw = plsc.load_gather(gt_der, [bi + 3 * MP])
                lgh = plsc.load_gather(gt_der, [bi + 4 * MP])
                clsf = plsc.load_gather(gt_der, [bi + 5 * MP])
                tcx = anc_der[1, pl.ds(o, L)]
                tcy = anc_der[2, pl.ds(o, L)]
                lw = anc_der[3, pl.ds(o, L)]
                lh = anc_der[4, pl.ds(o, L)]
                ridx = iota5 + o * 5
                plsc.store_scatter(reg_buf, [ridx],
                                   jnp.where(pos, gcx - tcx, 0.0))
                plsc.store_scatter(reg_buf, [ridx + 1],
                                   jnp.where(pos, gcy - tcy, 0.0))
                plsc.store_scatter(reg_buf, [ridx + 2],
                                   jnp.where(pos, lgw - lw, 0.0))
                plsc.store_scatter(reg_buf, [ridx + 3],
                                   jnp.where(pos, lgh - lh, 0.0))
                plsc.store_scatter(reg_buf, [ridx + 4], zeros)
                cidx = (clsf.astype(jnp.int32) + iota20) + o * N_CLASSES
                plsc.store_scatter(cls_buf, [cidx], ones, mask=pos)

        return run

    group4 = make_group(GROUP)

    def batch_body(b, _):
        def zero_chunk(i, _):
            o = i * (20 * L)
            for u in range(20):
                cls_buf[pl.ds(o + u * L, L)] = zeros
            return 0

        lax.fori_loop(0, NV, zero_chunk, 0)

        def group_body(g, _):
            group4(b, g * (GROUP * L))
            return 0

        lax.fori_loop(0, NGROUPS, group_body, 0)

        pltpu.sync_copy(pos_buf, pos_hbm.at[pl.ds(b * NAP + base, TN)])
        pltpu.sync_copy(
            cls_buf,
            cls_hbm.at[pl.ds((b * NAP + base) * N_CLASSES, TN * N_CLASSES)])
        pltpu.sync_copy(
            reg_buf, reg_hbm.at[pl.ds((b * NAP + base) * 5, TN * 5)])
        return 0

    lax.fori_loop(0, BATCH, batch_body, 0)


@functools.partial(
    pl.kernel,
    out_type=(
        jax.ShapeDtypeStruct((BATCH * NAP,), jnp.float32),
        jax.ShapeDtypeStruct((BATCH * NAP * N_CLASSES,), jnp.float32),
        jax.ShapeDtypeStruct((BATCH * NAP * 5,), jnp.float32),
    ),
    mesh=plsc.VectorSubcoreMesh(
        core_axis_name="c", subcore_axis_name="s",
        num_cores=NC, num_subcores=NS),
    compiler_params=pltpu.CompilerParams(needs_layout_passes=False),
    scratch_types=(
        pltpu.VMEM((4, TN), jnp.float32),
        pltpu.VMEM((5, TN), jnp.float32),
        pltpu.VMEM((BATCH * 6 * MP,), jnp.float32),
        pltpu.VMEM((BATCH * 6 * MP,), jnp.float32),
        pltpu.VMEM((TN,), jnp.float32),
        pltpu.VMEM((TN * N_CLASSES,), jnp.float32),
        pltpu.VMEM((TN * 5,), jnp.float32),
    ),
)
def _label_assign_sc(anc_hbm, gt_hbm, pos_hbm, cls_hbm, reg_hbm,
                     anc_raw, anc_der, gt_raw, gt_der,
                     pos_buf, cls_buf, reg_buf):
    _sc_body(anc_hbm, gt_hbm, pos_hbm, cls_hbm, reg_hbm,
             anc_raw, anc_der, gt_raw, gt_der, pos_buf, cls_buf, reg_buf)


def kernel(anchor, target, regressions, classifications):
    del regressions, classifications
    anc_t = jnp.zeros((4, NAP), jnp.float32).at[:, :N_ANCHORS].set(anchor.T)
    pad_row = jnp.array([0.0, 0.0, 1.0, 1.0, 0.0, 0.0], jnp.float32)
    tgt = jnp.concatenate(
        [target, jnp.broadcast_to(pad_row, (BATCH, MP - M_GT, 6))], axis=1)
    gt_t = jnp.transpose(tgt, (0, 2, 1)).reshape(-1)

    pos, cls, reg = _label_assign_sc(anc_t, gt_t)

    positive = pos.reshape(BATCH, NAP)[:, :N_ANCHORS] > 0.5
    cls_out = cls.reshape(BATCH, NAP, N_CLASSES)[:, :N_ANCHORS]
    reg_out = reg.reshape(BATCH, NAP, 5)[:, :N_ANCHORS]
    return positive, cls_out, reg_out


# speedup vs baseline: 1.1368x; 1.1109x over previous
import functools

import jax
import jax.numpy as jnp
from jax import lax
from jax.experimental import pallas as pl
from jax.experimental.pallas import tpu as pltpu
from jax.experimental.pallas import tpu_sc as plsc

N_ANCHORS = 20000
BATCH = 8
M_GT = 100
N_CLASSES = 20
POS_IOU_THR = 0.3

NC, NS, L = 2, 16, 16
NW = NC * NS
STRIDE = 624
TN = 656
MP = 112
NV = TN // L
GROUP = 4
NGROUPS = 10

_LN2 = 0.6931471805599453
_SQRT2 = 1.4142135623730951


def _vlog(x):
    bits = lax.bitcast_convert_type(x, jnp.int32)
    e = lax.shift_right_logical(bits, 23) - 127
    m = lax.bitcast_convert_type(
        (bits & 0x7FFFFF) | 0x3F800000, jnp.float32)
    big = m > _SQRT2
    m = jnp.where(big, m * 0.5, m)
    e = jnp.where(big, e + 1, e)
    z = (m - 1.0) / (m + 1.0)
    z2 = z * z
    p = ((z2 * (1.0 / 9.0) + (1.0 / 7.0)) * z2 + (1.0 / 5.0)) * z2 + (1.0 / 3.0)
    logm = 2.0 * z * (p * z2 + 1.0)
    return e.astype(jnp.float32) * _LN2 + logm


def _sc_body(anc_hbm, gt_hbm, pos_hbm, cls_hbm, reg_hbm,
             anc_raw, anc_der, gt_raw, gt_der, pos_buf, cls_buf, reg_buf):
    wid = lax.axis_index("s") * NC + lax.axis_index("c")
    base = wid * STRIDE

    pltpu.sync_copy(anc_hbm.at[:, pl.ds(base, TN)], anc_raw)
    pltpu.sync_copy(gt_hbm, gt_raw)

    iota = lax.iota(jnp.int32, L)
    iota20 = iota * N_CLASSES
    iota5 = iota * 5
    zeros = jnp.zeros((L,), jnp.float32)
    ones = jnp.ones((L,), jnp.float32)

    def anc_chunk(i, _):
        o = i * L
        x1 = anc_raw[0, pl.ds(o, L)]
        y1 = anc_raw[1, pl.ds(o, L)]
        x2 = anc_raw[2, pl.ds(o, L)]
        y2 = anc_raw[3, pl.ds(o, L)]
        dx = x2 - x1
        dy = y2 - y1
        anc_der[0, pl.ds(o, L)] = dx * dy
        ex_w = jnp.maximum(dx, 1.0)
        ex_h = jnp.maximum(dy, 1.0)
        cx = x1 + 0.5 * ex_w
        cy = y1 + 0.5 * ex_h
        anc_der[1, pl.ds(o, L)] = cx.astype(jnp.int32).astype(jnp.float32)
        anc_der[2, pl.ds(o, L)] = cy.astype(jnp.int32).astype(jnp.float32)
        anc_der[3, pl.ds(o, L)] = _vlog(ex_w)
        anc_der[4, pl.ds(o, L)] = _vlog(ex_h)
        return 0

    lax.fori_loop(0, NV, anc_chunk, 0)

    def gt_batch(b, _):
        gbase = b * 6 * MP

        def gt_chunk(p, _):
            o = p * L
            x1 = gt_raw[pl.ds(gbase + o, L)]
            y1 = gt_raw[pl.ds(gbase + MP + o, L)]
            x2 = gt_raw[pl.ds(gbase + 2 * MP + o, L)]
            y2 = gt_raw[pl.ds(gbase + 3 * MP + o, L)]
            clsf = gt_raw[pl.ds(gbase + 4 * MP + o, L)]
            dx = x2 - x1
            dy = y2 - y1
            gt_der[pl.ds(gbase + o, L)] = dx * dy
            gw = jnp.maximum(dx, 1.0)
            gh = jnp.maximum(dy, 1.0)
            gt_der[pl.ds(gbase + MP + o, L)] = x1 + 0.5 * gw
            gt_der[pl.ds(gbase + 2 * MP + o, L)] = y1 + 0.5 * gh
            gt_der[pl.ds(gbase + 3 * MP + o, L)] = _vlog(gw)
            gt_der[pl.ds(gbase + 4 * MP + o, L)] = _vlog(gh)
            gt_der[pl.ds(gbase + 5 * MP + o, L)] = clsf
            return 0

        lax.fori_loop(0, MP // L, gt_chunk, 0)
        return 0

    lax.fori_loop(0, BATCH, gt_batch, 0)

    def make_group(nv):
        def run(b, gb):
            gbase = b * 6 * MP
            ax1 = [anc_raw[0, pl.ds(gb + v * L, L)] for v in range(nv)]
            ay1 = [anc_raw[1, pl.ds(gb + v * L, L)] for v in range(nv)]
            ax2 = [anc_raw[2, pl.ds(gb + v * L, L)] for v in range(nv)]
            ay2 = [anc_raw[3, pl.ds(gb + v * L, L)] for v in range(nv)]
            aar = [anc_der[0, pl.ds(gb + v * L, L)] for v in range(nv)]

            def jbody(j, carry):
                bests, bestis = carry
                jv = jnp.full((L,), j, jnp.int32)
                idx0 = jv + gbase
                gx1 = plsc.load_gather(gt_raw, [idx0])
                gy1 = plsc.load_gather(gt_raw, [idx0 + MP])
                gx2 = plsc.load_gather(gt_raw, [idx0 + 2 * MP])
                gy2 = plsc.load_gather(gt_raw, [idx0 + 3 * MP])
                ab = plsc.load_gather(gt_der, [idx0])
                nb, ni = [], []
                for v in range(nv):
                    ltx = jnp.maximum(ax1[v], gx1)
                    lty = jnp.maximum(ay1[v], gy1)
                    rbx = jnp.minimum(ax2[v], gx2)
                    rby = jnp.minimum(ay2[v], gy2)
                    wx = jnp.maximum(rbx - ltx, 0.0)
                    wy = jnp.maximum(rby - lty, 0.0)
                    inter = wx * wy
                    den = (aar[v] + ab) - inter + 1e-10
                    iou = inter / den
                    better = iou > bests[v]
                    nb.append(jnp.where(better, iou, bests[v]))
                    ni.append(jnp.where(better, jv, bestis[v]))
                return tuple(nb), tuple(ni)

            init = (tuple(jnp.full((L,), -1.0, jnp.float32) for _ in range(nv)),
                    tuple(jnp.zeros((L,), jnp.int32) for _ in range(nv)))
            bests, bestis = lax.fori_loop(0, M_GT, jbody, init)

            for v in range(nv):
                o = gb + v * L
                pos = bests[v] >= POS_IOU_THR
                pos_buf[pl.ds(o, L)] = jnp.where(pos, ones, zeros)
                bi = bestis[v] + gbase
                gcx = plsc.load_gather(gt_der, [bi + MP])
                gcy = plsc.load_gather(gt_der, [bi + 2 * MP])
                lgw = plsc.load_gather(gt_der, [bi + 3 * MP])
                lgh = plsc.load_gather(gt_der, [bi + 4 * MP])
                clsf = plsc.load_gather(gt_der, [bi + 5 * MP])
                tcx = anc_der[1, pl.ds(o, L)]
                tcy = anc_der[2, pl.ds(o, L)]
                lw = anc_der[3, pl.ds(o, L)]
                lh = anc_der[4, pl.ds(o, L)]
                ridx = iota5 + o * 5
                plsc.store_scatter(reg_buf, [ridx],
                                   jnp.where(pos, gcx - tcx, 0.0))
                plsc.store_scatter(reg_buf, [ridx + 1],
                                   jnp.where(pos, gcy - tcy, 0.0))
                plsc.store_scatter(reg_buf, [ridx + 2],
                                   jnp.where(pos, lgw - lw, 0.0))
                plsc.store_scatter(reg_buf, [ridx + 3],
                                   jnp.where(pos, lgh - lh, 0.0))
                plsc.store_scatter(reg_buf, [ridx + 4], zeros)
                cidx = (clsf.astype(jnp.int32) + iota20) + o * N_CLASSES
                plsc.store_scatter(cls_buf, [cidx], ones, mask=pos)

        return run

    group4 = make_group(GROUP)
    group1 = make_group(1)

    def batch_body(b, _):
        def zero_chunk(i, _):
            o = i * (20 * L)
            for u in range(20):
                cls_buf[pl.ds(o + u * L, L)] = zeros
            return 0

        lax.fori_loop(0, NV, zero_chunk, 0)

        def group_body(g, _):
            group4(b, g * (GROUP * L))
            return 0

        lax.fori_loop(0, NGROUPS, group_body, 0)
        group1(b, NGROUPS * GROUP * L)

        pltpu.sync_copy(pos_buf, pos_hbm.at[pl.ds(b * N_ANCHORS + base, TN)])
        pltpu.sync_copy(
            cls_buf,
            cls_hbm.at[pl.ds((b * N_ANCHORS + base) * N_CLASSES,
                             TN * N_CLASSES)])
        pltpu.sync_copy(
            reg_buf, reg_hbm.at[pl.ds((b * N_ANCHORS + base) * 5, TN * 5)])
        return 0

    lax.fori_loop(0, BATCH, batch_body, 0)


@functools.partial(
    pl.kernel,
    out_type=(
        jax.ShapeDtypeStruct((BATCH * N_ANCHORS,), jnp.float32),
        jax.ShapeDtypeStruct((BATCH * N_ANCHORS * N_CLASSES,), jnp.float32),
        jax.ShapeDtypeStruct((BATCH * N_ANCHORS * 5,), jnp.float32),
    ),
    mesh=plsc.VectorSubcoreMesh(
        core_axis_name="c", subcore_axis_name="s",
        num_cores=NC, num_subcores=NS),
    compiler_params=pltpu.CompilerParams(
        needs_layout_passes=False, use_tc_tiling_on_sc=False),
    scratch_types=(
        pltpu.VMEM((4, TN), jnp.float32),
        pltpu.VMEM((5, TN), jnp.float32),
        pltpu.VMEM((BATCH * 6 * MP,), jnp.float32),
        pltpu.VMEM((BATCH * 6 * MP,), jnp.float32),
        pltpu.VMEM((TN,), jnp.float32),
        pltpu.VMEM((TN * N_CLASSES,), jnp.float32),
        pltpu.VMEM((TN * 5,), jnp.float32),
    ),
)
def _label_assign_sc(anc_hbm, gt_hbm, pos_hbm, cls_hbm, reg_hbm,
                     anc_raw, anc_der, gt_raw, gt_der,
                     pos_buf, cls_buf, reg_buf):
    _sc_body(anc_hbm, gt_hbm, pos_hbm, cls_hbm, reg_hbm,
             anc_raw, anc_der, gt_raw, gt_der, pos_buf, cls_buf, reg_buf)


def kernel(anchor, target, regressions, classifications):
    del regressions, classifications
    anc_t = anchor.T
    pad_row = jnp.array([0.0, 0.0, 1.0, 1.0, 0.0, 0.0], jnp.float32)
    tgt = jnp.concatenate(
        [target, jnp.broadcast_to(pad_row, (BATCH, MP - M_GT, 6))], axis=1)
    gt_t = jnp.transpose(tgt, (0, 2, 1)).reshape(-1)

    pos, cls, reg = _label_assign_sc(anc_t, gt_t)

    positive = pos.reshape(BATCH, N_ANCHORS) > 0.5
    cls_out = cls.reshape(BATCH, N_ANCHORS, N_CLASSES)
    reg_out = reg.reshape(BATCH, N_ANCHORS, 5)
    return positive, cls_out, reg_out


# speedup vs baseline: 1.2106x; 1.0650x over previous
import functools

import jax
import jax.numpy as jnp
from jax import lax
from jax.experimental import pallas as pl
from jax.experimental.pallas import tpu as pltpu
from jax.experimental.pallas import tpu_sc as plsc

N_ANCHORS = 20000
BATCH = 8
M_GT = 100
N_CLASSES = 20
POS_IOU_THR = 0.3

NC, NS, L = 2, 16, 16
NW = NC * NS
STRIDE = 624
TN = 656
MP = 112
NV = TN // L
GROUP = 4
NGROUPS = 10

_LN2 = 0.6931471805599453
_SQRT2 = 1.4142135623730951


def _vlog(x):
    bits = lax.bitcast_convert_type(x, jnp.int32)
    e = lax.shift_right_logical(bits, 23) - 127
    m = lax.bitcast_convert_type(
        (bits & 0x7FFFFF) | 0x3F800000, jnp.float32)
    big = m > _SQRT2
    m = jnp.where(big, m * 0.5, m)
    e = jnp.where(big, e + 1, e)
    z = (m - 1.0) / (m + 1.0)
    z2 = z * z
    p = ((z2 * (1.0 / 9.0) + (1.0 / 7.0)) * z2 + (1.0 / 5.0)) * z2 + (1.0 / 3.0)
    logm = 2.0 * z * (p * z2 + 1.0)
    return e.astype(jnp.float32) * _LN2 + logm


def _sc_body(anc_hbm, gt_hbm, pos_hbm, cls_hbm, reg_hbm,
             anc_raw, anc_der, gt_raw, gt_der, pos_buf, cls_buf, reg_buf):
    wid = lax.axis_index("s") * NC + lax.axis_index("c")
    base = wid * STRIDE

    pltpu.sync_copy(anc_hbm.at[:, pl.ds(base, TN)], anc_raw)
    pltpu.sync_copy(gt_hbm, gt_raw)

    iota = lax.iota(jnp.int32, L)
    zeros = jnp.zeros((L,), jnp.float32)
    ones = jnp.ones((L,), jnp.float32)
    icol0 = jnp.zeros((L,), jnp.int32)
    icol1 = icol0 + 1
    icol2 = icol0 + 2
    icol3 = icol0 + 3
    icol4 = icol0 + 4
    ccols = [icol0 + c for c in range(N_CLASSES)]

    def anc_chunk(i, _):
        o = i * L
        x1 = anc_raw[0, pl.ds(o, L)]
        y1 = anc_raw[1, pl.ds(o, L)]
        x2 = anc_raw[2, pl.ds(o, L)]
        y2 = anc_raw[3, pl.ds(o, L)]
        dx = x2 - x1
        dy = y2 - y1
        anc_der[0, pl.ds(o, L)] = dx * dy
        ex_w = jnp.maximum(dx, 1.0)
        ex_h = jnp.maximum(dy, 1.0)
        cx = x1 + 0.5 * ex_w
        cy = y1 + 0.5 * ex_h
        anc_der[1, pl.ds(o, L)] = cx.astype(jnp.int32).astype(jnp.float32)
        anc_der[2, pl.ds(o, L)] = cy.astype(jnp.int32).astype(jnp.float32)
        anc_der[3, pl.ds(o, L)] = _vlog(ex_w)
        anc_der[4, pl.ds(o, L)] = _vlog(ex_h)
        return 0

    lax.fori_loop(0, NV, anc_chunk, 0)

    def gt_batch(b, _):
        gbase = b * 6 * MP

        def gt_chunk(p, _):
            o = p * L
            x1 = gt_raw[pl.ds(gbase + o, L)]
            y1 = gt_raw[pl.ds(gbase + MP + o, L)]
            x2 = gt_raw[pl.ds(gbase + 2 * MP + o, L)]
            y2 = gt_raw[pl.ds(gbase + 3 * MP + o, L)]
            clsf = gt_raw[pl.ds(gbase + 4 * MP + o, L)]
            dx = x2 - x1
            dy = y2 - y1
            gt_der[pl.ds(gbase + o, L)] = dx * dy
            gw = jnp.maximum(dx, 1.0)
            gh = jnp.maximum(dy, 1.0)
            gt_der[pl.ds(gbase + MP + o, L)] = x1 + 0.5 * gw
            gt_der[pl.ds(gbase + 2 * MP + o, L)] = y1 + 0.5 * gh
            gt_der[pl.ds(gbase + 3 * MP + o, L)] = _vlog(gw)
            gt_der[pl.ds(gbase + 4 * MP + o, L)] = _vlog(gh)
            gt_der[pl.ds(gbase + 5 * MP + o, L)] = clsf
            return 0

        lax.fori_loop(0, MP // L, gt_chunk, 0)
        return 0

    lax.fori_loop(0, BATCH, gt_batch, 0)

    def make_group(nv):
        def run(b, gb):
            gbase = b * 6 * MP
            ax1 = [anc_raw[0, pl.ds(gb + v * L, L)] for v in range(nv)]
            ay1 = [anc_raw[1, pl.ds(gb + v * L, L)] for v in range(nv)]
            ax2 = [anc_raw[2, pl.ds(gb + v * L, L)] for v in range(nv)]
            ay2 = [anc_raw[3, pl.ds(gb + v * L, L)] for v in range(nv)]
            aar = [anc_der[0, pl.ds(gb + v * L, L)] for v in range(nv)]

            def jbody(j, carry):
                bests, bestis = carry
                jv = jnp.full((L,), j, jnp.int32)
                idx0 = jv + gbase
                gx1 = plsc.load_gather(gt_raw, [idx0])
                gy1 = plsc.load_gather(gt_raw, [idx0 + MP])
                gx2 = plsc.load_gather(gt_raw, [idx0 + 2 * MP])
                gy2 = plsc.load_gather(gt_raw, [idx0 + 3 * MP])
                ab = plsc.load_gather(gt_der, [idx0])
                nb, ni = [], []
                for v in range(nv):
                    ltx = jnp.maximum(ax1[v], gx1)
                    lty = jnp.maximum(ay1[v], gy1)
                    rbx = jnp.minimum(ax2[v], gx2)
                    rby = jnp.minimum(ay2[v], gy2)
                    wx = jnp.maximum(rbx - ltx, 0.0)
                    wy = jnp.maximum(rby - lty, 0.0)
                    inter = wx * wy
                    den = (aar[v] + ab) - inter + 1e-10
                    iou = inter / den
                    better = iou > bests[v]
                    nb.append(jnp.where(better, iou, bests[v]))
                    ni.append(jnp.where(better, jv, bestis[v]))
                return tuple(nb), tuple(ni)

            init = (tuple(jnp.full((L,), -1.0, jnp.float32) for _ in range(nv)),
                    tuple(jnp.zeros((L,), jnp.int32) for _ in range(nv)))
            bests, bestis = lax.fori_loop(0, M_GT, jbody, init)

            for v in range(nv):
                o = gb + v * L
                pos = bests[v] >= POS_IOU_THR
                pos_buf[pl.ds(o, L)] = jnp.where(pos, ones, zeros)
                bi = bestis[v] + gbase
                gcx = plsc.load_gather(gt_der, [bi + MP])
                gcy = plsc.load_gather(gt_der, [bi + 2 * MP])
                lgw = plsc.load_gather(gt_der, [bi + 3 * MP])
                lgh = plsc.load_gather(gt_der, [bi + 4 * MP])
                clsf = plsc.load_gather(gt_der, [bi + 5 * MP])
                tcx = anc_der[1, pl.ds(o, L)]
                tcy = anc_der[2, pl.ds(o, L)]
                lw = anc_der[3, pl.ds(o, L)]
                lh = anc_der[4, pl.ds(o, L)]
                rows = iota + o
                plsc.store_scatter(reg_buf, [rows, icol0],
                                   jnp.where(pos, gcx - tcx, 0.0))
                plsc.store_scatter(reg_buf, [rows, icol1],
                                   jnp.where(pos, gcy - tcy, 0.0))
                plsc.store_scatter(reg_buf, [rows, icol2],
                                   jnp.where(pos, lgw - lw, 0.0))
                plsc.store_scatter(reg_buf, [rows, icol3],
                                   jnp.where(pos, lgh - lh, 0.0))
                plsc.store_scatter(reg_buf, [rows, icol4], zeros)
                cidx = clsf.astype(jnp.int32)
                plsc.store_scatter(cls_buf, [rows, cidx], ones, mask=pos)

        return run

    group4 = make_group(GROUP)
    group1 = make_group(1)

    def batch_body(b, _):
        def zero_chunk(i, _):
            rows = iota + i * L
            for c in range(N_CLASSES):
                plsc.store_scatter(cls_buf, [rows, ccols[c]], zeros)
            return 0

        lax.fori_loop(0, NV, zero_chunk, 0)

        def group_body(g, _):
            group4(b, g * (GROUP * L))
            return 0

        lax.fori_loop(0, NGROUPS, group_body, 0)
        group1(b, NGROUPS * GROUP * L)

        pltpu.sync_copy(pos_buf, pos_hbm.at[b, pl.ds(base, TN)])
        pltpu.sync_copy(cls_buf, cls_hbm.at[b, pl.ds(base, TN)])
        pltpu.sync_copy(reg_buf, reg_hbm.at[b, pl.ds(base, TN)])
        return 0

    lax.fori_loop(0, BATCH, batch_body, 0)


@functools.partial(
    pl.kernel,
    out_type=(
        jax.ShapeDtypeStruct((BATCH, N_ANCHORS), jnp.float32),
        jax.ShapeDtypeStruct((BATCH, N_ANCHORS, N_CLASSES), jnp.float32),
        jax.ShapeDtypeStruct((BATCH, N_ANCHORS, 5), jnp.float32),
    ),
    mesh=plsc.VectorSubcoreMesh(
        core_axis_name="c", subcore_axis_name="s",
        num_cores=NC, num_subcores=NS),
    compiler_params=pltpu.CompilerParams(
        needs_layout_passes=False, use_tc_tiling_on_sc=False),
    scratch_types=(
        pltpu.VMEM((4, TN), jnp.float32),
        pltpu.VMEM((5, TN), jnp.float32),
        pltpu.VMEM((BATCH * 6 * MP,), jnp.float32),
        pltpu.VMEM((BATCH * 6 * MP,), jnp.float32),
        pltpu.VMEM((TN,), jnp.float32),
        pltpu.VMEM((TN, N_CLASSES), jnp.float32),
        pltpu.VMEM((TN, 5), jnp.float32),
    ),
)
def _label_assign_sc(anc_hbm, gt_hbm, pos_hbm, cls_hbm, reg_hbm,
                     anc_raw, anc_der, gt_raw, gt_der,
                     pos_buf, cls_buf, reg_buf):
    _sc_body(anc_hbm, gt_hbm, pos_hbm, cls_hbm, reg_hbm,
             anc_raw, anc_der, gt_raw, gt_der, pos_buf, cls_buf, reg_buf)


def kernel(anchor, target, regressions, classifications):
    del regressions, classifications
    anc_t = anchor.T
    pad_row = jnp.array([0.0, 0.0, 1.0, 1.0, 0.0, 0.0], jnp.float32)
    tgt = jnp.concatenate(
        [target, jnp.broadcast_to(pad_row, (BATCH, MP - M_GT, 6))], axis=1)
    gt_t = jnp.transpose(tgt, (0, 2, 1)).reshape(-1)

    pos, cls, reg = _label_assign_sc(anc_t, gt_t)

    positive = pos > 0.5
    return positive, cls, reg


# speedup vs baseline: 3.5044x; 2.8947x over previous
import functools

import jax
import jax.numpy as jnp
from jax import lax
from jax.experimental import pallas as pl
from jax.experimental.pallas import tpu as pltpu
from jax.experimental.pallas import tpu_sc as plsc

N_ANCHORS = 20000
BATCH = 8
M_GT = 100
N_CLASSES = 20
POS_IOU_THR = 0.3

NC, NS, L = 2, 16, 16
NW = NC * NS
STRIDE = 624
TN = 656
MP = 112
NV = TN // L
GROUP = 4
NGROUPS = 10

_LN2 = 0.6931471805599453
_SQRT2 = 1.4142135623730951


def _vlog(x):
    bits = lax.bitcast_convert_type(x, jnp.int32)
    e = lax.shift_right_logical(bits, 23) - 127
    m = lax.bitcast_convert_type(
        (bits & 0x7FFFFF) | 0x3F800000, jnp.float32)
    big = m > _SQRT2
    m = jnp.where(big, m * 0.5, m)
    e = jnp.where(big, e + 1, e)
    z = (m - 1.0) / (m + 1.0)
    z2 = z * z
    p = ((z2 * (1.0 / 9.0) + (1.0 / 7.0)) * z2 + (1.0 / 5.0)) * z2 + (1.0 / 3.0)
    logm = 2.0 * z * (p * z2 + 1.0)
    return e.astype(jnp.float32) * _LN2 + logm


def _sc_body(anc_hbm, gt_hbm, pos_hbm, cls_hbm, reg_hbm,
             anc_raw, anc_der, gt_raw, gt_der, pos_buf, cls_buf, reg_buf):
    wid = lax.axis_index("s") * NC + lax.axis_index("c")
    base = wid * STRIDE

    pltpu.sync_copy(anc_hbm.at[:, pl.ds(base, TN)], anc_raw)
    pltpu.sync_copy(gt_hbm, gt_raw)

    iota = lax.iota(jnp.int32, L)
    zeros = jnp.zeros((L,), jnp.float32)
    ones = jnp.ones((L,), jnp.float32)

    def anc_chunk(i, _):
        o = i * L
        x1 = anc_raw[0, pl.ds(o, L)]
        y1 = anc_raw[1, pl.ds(o, L)]
        x2 = anc_raw[2, pl.ds(o, L)]
        y2 = anc_raw[3, pl.ds(o, L)]
        dx = x2 - x1
        dy = y2 - y1
        anc_der[0, pl.ds(o, L)] = dx * dy
        ex_w = jnp.maximum(dx, 1.0)
        ex_h = jnp.maximum(dy, 1.0)
        cx = x1 + 0.5 * ex_w
        cy = y1 + 0.5 * ex_h
        anc_der[1, pl.ds(o, L)] = cx.astype(jnp.int32).astype(jnp.float32)
        anc_der[2, pl.ds(o, L)] = cy.astype(jnp.int32).astype(jnp.float32)
        anc_der[3, pl.ds(o, L)] = _vlog(ex_w)
        anc_der[4, pl.ds(o, L)] = _vlog(ex_h)
        return 0

    lax.fori_loop(0, NV, anc_chunk, 0)

    def gt_batch(b, _):
        gbase = b * 6 * MP

        def gt_chunk(p, _):
            o = p * L
            x1 = gt_raw[pl.ds(gbase + o, L)]
            y1 = gt_raw[pl.ds(gbase + MP + o, L)]
            x2 = gt_raw[pl.ds(gbase + 2 * MP + o, L)]
            y2 = gt_raw[pl.ds(gbase + 3 * MP + o, L)]
            clsf = gt_raw[pl.ds(gbase + 4 * MP + o, L)]
            dx = x2 - x1
            dy = y2 - y1
            gt_der[pl.ds(gbase + o, L)] = dx * dy
            gw = jnp.maximum(dx, 1.0)
            gh = jnp.maximum(dy, 1.0)
            gt_der[pl.ds(gbase + MP + o, L)] = x1 + 0.5 * gw
            gt_der[pl.ds(gbase + 2 * MP + o, L)] = y1 + 0.5 * gh
            gt_der[pl.ds(gbase + 3 * MP + o, L)] = _vlog(gw)
            gt_der[pl.ds(gbase + 4 * MP + o, L)] = _vlog(gh)
            gt_der[pl.ds(gbase + 5 * MP + o, L)] = clsf
            return 0

        lax.fori_loop(0, MP // L, gt_chunk, 0)
        return 0

    lax.fori_loop(0, BATCH, gt_batch, 0)

    def make_group(nv):
        def run(b, gb):
            gbase = b * 6 * MP
            ax1 = [anc_raw[0, pl.ds(gb + v * L, L)] for v in range(nv)]
            ay1 = [anc_raw[1, pl.ds(gb + v * L, L)] for v in range(nv)]
            ax2 = [anc_raw[2, pl.ds(gb + v * L, L)] for v in range(nv)]
            ay2 = [anc_raw[3, pl.ds(gb + v * L, L)] for v in range(nv)]
            aar = [anc_der[0, pl.ds(gb + v * L, L)] for v in range(nv)]

            def jbody(j, carry):
                bests, bestis = carry
                jv = jnp.full((L,), j, jnp.int32)
                idx0 = jv + gbase
                gx1 = plsc.load_gather(gt_raw, [idx0])
                gy1 = plsc.load_gather(gt_raw, [idx0 + MP])
                gx2 = plsc.load_gather(gt_raw, [idx0 + 2 * MP])
                gy2 = plsc.load_gather(gt_raw, [idx0 + 3 * MP])
                ab = plsc.load_gather(gt_der, [idx0])
                nb, ni = [], []
                for v in range(nv):
                    ltx = jnp.maximum(ax1[v], gx1)
                    lty = jnp.maximum(ay1[v], gy1)
                    rbx = jnp.minimum(ax2[v], gx2)
                    rby = jnp.minimum(ay2[v], gy2)
                    wx = jnp.maximum(rbx - ltx, 0.0)
                    wy = jnp.maximum(rby - lty, 0.0)
                    inter = wx * wy
                    den = (aar[v] + ab) - inter + 1e-10
                    iou = inter / den
                    better = iou > bests[v]
                    nb.append(jnp.where(better, iou, bests[v]))
                    ni.append(jnp.where(better, jv, bestis[v]))
                return tuple(nb), tuple(ni)

            init = (tuple(jnp.full((L,), -1.0, jnp.float32) for _ in range(nv)),
                    tuple(jnp.zeros((L,), jnp.int32) for _ in range(nv)))
            bests, bestis = lax.fori_loop(0, M_GT, jbody, init)

            for v in range(nv):
                o = gb + v * L
                pos = bests[v] >= POS_IOU_THR
                pos_buf[pl.ds(o, L)] = jnp.where(pos, ones, zeros)
                bi = bestis[v] + gbase
                gcx = plsc.load_gather(gt_der, [bi + MP])
                gcy = plsc.load_gather(gt_der, [bi + 2 * MP])
                lgw = plsc.load_gather(gt_der, [bi + 3 * MP])
                lgh = plsc.load_gather(gt_der, [bi + 4 * MP])
                clsf = plsc.load_gather(gt_der, [bi + 5 * MP])
                tcx = anc_der[1, pl.ds(o, L)]
                tcy = anc_der[2, pl.ds(o, L)]
                lw = anc_der[3, pl.ds(o, L)]
                lh = anc_der[4, pl.ds(o, L)]
                rows = iota + o
                reg_buf[0, pl.ds(o, L)] = jnp.where(pos, gcx - tcx, 0.0)
                reg_buf[1, pl.ds(o, L)] = jnp.where(pos, gcy - tcy, 0.0)
                reg_buf[2, pl.ds(o, L)] = jnp.where(pos, lgw - lw, 0.0)
                reg_buf[3, pl.ds(o, L)] = jnp.where(pos, lgh - lh, 0.0)
                reg_buf[4, pl.ds(o, L)] = zeros
                cidx = clsf.astype(jnp.int32)
                plsc.store_scatter(cls_buf, [cidx, rows], ones, mask=pos)

        return run

    group4 = make_group(GROUP)
    group1 = make_group(1)

    def batch_body(b, _):
        def zero_chunk(i, _):
            o = i * L
            for c in range(N_CLASSES):
                cls_buf[c, pl.ds(o, L)] = zeros
            return 0

        lax.fori_loop(0, NV, zero_chunk, 0)

        def group_body(g, _):
            group4(b, g * (GROUP * L))
            return 0

        lax.fori_loop(0, NGROUPS, group_body, 0)
        group1(b, NGROUPS * GROUP * L)

        pltpu.sync_copy(pos_buf, pos_hbm.at[b, pl.ds(base, TN)])
        pltpu.sync_copy(cls_buf, cls_hbm.at[:, b, pl.ds(base, TN)])
        pltpu.sync_copy(reg_buf, reg_hbm.at[:, b, pl.ds(base, TN)])
        return 0

    lax.fori_loop(0, BATCH, batch_body, 0)


@functools.partial(
    pl.kernel,
    out_type=(
        jax.ShapeDtypeStruct((BATCH, N_ANCHORS), jnp.float32),
        jax.ShapeDtypeStruct((N_CLASSES, BATCH, N_ANCHORS), jnp.float32),
        jax.ShapeDtypeStruct((5, BATCH, N_ANCHORS), jnp.float32),
    ),
    mesh=plsc.VectorSubcoreMesh(
        core_axis_name="c", subcore_axis_name="s",
        num_cores=NC, num_subcores=NS),
    compiler_params=pltpu.CompilerParams(
        needs_layout_passes=False, use_tc_tiling_on_sc=False),
    scratch_types=(
        pltpu.VMEM((4, TN), jnp.float32),
        pltpu.VMEM((5, TN), jnp.float32),
        pltpu.VMEM((BATCH * 6 * MP,), jnp.float32),
        pltpu.VMEM((BATCH * 6 * MP,), jnp.float32),
        pltpu.VMEM((TN,), jnp.float32),
        pltpu.VMEM((N_CLASSES, TN), jnp.float32),
        pltpu.VMEM((5, TN), jnp.float32),
    ),
)
def _label_assign_sc(anc_hbm, gt_hbm, pos_hbm, cls_hbm, reg_hbm,
                     anc_raw, anc_der, gt_raw, gt_der,
                     pos_buf, cls_buf, reg_buf):
    _sc_body(anc_hbm, gt_hbm, pos_hbm, cls_hbm, reg_hbm,
             anc_raw, anc_der, gt_raw, gt_der, pos_buf, cls_buf, reg_buf)


def kernel(anchor, target, regressions, classifications):
    del regressions, classifications
    anc_t = anchor.T
    pad_row = jnp.array([0.0, 0.0, 1.0, 1.0, 0.0, 0.0], jnp.float32)
    tgt = jnp.concatenate(
        [target, jnp.broadcast_to(pad_row, (BATCH, MP - M_GT, 6))], axis=1)
    gt_t = jnp.transpose(tgt, (0, 2, 1)).reshape(-1)

    pos, cls, reg = _label_assign_sc(anc_t, gt_t)

    positive = pos > 0.5
    cls_out = jnp.transpose(cls, (1, 2, 0))
    reg_out = jnp.transpose(reg, (1, 2, 0))
    return positive, cls_out, reg_out


# speedup vs baseline: 3.6561x; 1.0433x over previous
import functools

import jax
import jax.numpy as jnp
from jax import lax
from jax.experimental import pallas as pl
from jax.experimental.pallas import tpu as pltpu
from jax.experimental.pallas import tpu_sc as plsc

N_ANCHORS = 20000
BATCH = 8
M_GT = 100
N_CLASSES = 20
POS_IOU_THR = 0.3

NC, NS, L = 2, 16, 16
NW = NC * NS
STRIDE = 624
TN = 656
MP = 112
NV = TN // L
GROUP = 4
NGROUPS = 9

_LN2 = 0.6931471805599453
_SQRT2 = 1.4142135623730951


def _vlog(x):
    bits = lax.bitcast_convert_type(x, jnp.int32)
    e = lax.shift_right_logical(bits, 23) - 127
    m = lax.bitcast_convert_type(
        (bits & 0x7FFFFF) | 0x3F800000, jnp.float32)
    big = m > _SQRT2
    m = jnp.where(big, m * 0.5, m)
    e = jnp.where(big, e + 1, e)
    z = (m - 1.0) / (m + 1.0)
    z2 = z * z
    p = ((z2 * (1.0 / 9.0) + (1.0 / 7.0)) * z2 + (1.0 / 5.0)) * z2 + (1.0 / 3.0)
    logm = 2.0 * z * (p * z2 + 1.0)
    return e.astype(jnp.float32) * _LN2 + logm


def _sc_body(anc_hbm, gt_hbm, pos_hbm, cls_hbm, reg_hbm,
             anc_raw, anc_der, gt_raw, gt_der, pos_buf, cls_buf, reg_buf,
             oc_buf, sems):
    wid = lax.axis_index("s") * NC + lax.axis_index("c")
    base = wid * STRIDE

    pltpu.sync_copy(anc_hbm.at[:, pl.ds(base, TN)], anc_raw)
    pltpu.sync_copy(gt_hbm, gt_raw)

    iota = lax.iota(jnp.int32, L)
    zeros = jnp.zeros((L,), jnp.float32)
    ones = jnp.ones((L,), jnp.float32)

    def anc_chunk(i, _):
        o = i * L
        x1 = anc_raw[0, pl.ds(o, L)]
        y1 = anc_raw[1, pl.ds(o, L)]
        x2 = anc_raw[2, pl.ds(o, L)]
        y2 = anc_raw[3, pl.ds(o, L)]
        dx = x2 - x1
        dy = y2 - y1
        anc_der[0, pl.ds(o, L)] = dx * dy
        ex_w = jnp.maximum(dx, 1.0)
        ex_h = jnp.maximum(dy, 1.0)
        cx = x1 + 0.5 * ex_w
        cy = y1 + 0.5 * ex_h
        anc_der[1, pl.ds(o, L)] = cx.astype(jnp.int32).astype(jnp.float32)
        anc_der[2, pl.ds(o, L)] = cy.astype(jnp.int32).astype(jnp.float32)
        anc_der[3, pl.ds(o, L)] = _vlog(ex_w)
        anc_der[4, pl.ds(o, L)] = _vlog(ex_h)
        return 0

    lax.fori_loop(0, NV, anc_chunk, 0)

    def gt_batch(b, _):
        gbase = b * 6 * MP

        def gt_chunk(p, _):
            o = p * L
            x1 = gt_raw[pl.ds(gbase + o, L)]
            y1 = gt_raw[pl.ds(gbase + MP + o, L)]
            x2 = gt_raw[pl.ds(gbase + 2 * MP + o, L)]
            y2 = gt_raw[pl.ds(gbase + 3 * MP + o, L)]
            clsf = gt_raw[pl.ds(gbase + 4 * MP + o, L)]
            dx = x2 - x1
            dy = y2 - y1
            gt_der[pl.ds(gbase + o, L)] = dx * dy
            gw = jnp.maximum(dx, 1.0)
            gh = jnp.maximum(dy, 1.0)
            gt_der[pl.ds(gbase + MP + o, L)] = x1 + 0.5 * gw
            gt_der[pl.ds(gbase + 2 * MP + o, L)] = y1 + 0.5 * gh
            gt_der[pl.ds(gbase + 3 * MP + o, L)] = _vlog(gw)
            gt_der[pl.ds(gbase + 4 * MP + o, L)] = _vlog(gh)
            gt_der[pl.ds(gbase + 5 * MP + o, L)] = clsf
            return 0

        lax.fori_loop(0, MP // L, gt_chunk, 0)
        return 0

    lax.fori_loop(0, BATCH, gt_batch, 0)

    def make_group(nv):
        def run(b, bi, gb):
            gbase = b * 6 * MP
            ax1 = [anc_raw[0, pl.ds(gb + v * L, L)] for v in range(nv)]
            ay1 = [anc_raw[1, pl.ds(gb + v * L, L)] for v in range(nv)]
            ax2 = [anc_raw[2, pl.ds(gb + v * L, L)] for v in range(nv)]
            ay2 = [anc_raw[3, pl.ds(gb + v * L, L)] for v in range(nv)]
            aar = [anc_der[0, pl.ds(gb + v * L, L)] for v in range(nv)]

            def jbody(j, carry):
                bests, bestis = carry
                jv = jnp.full((L,), j, jnp.int32)
                idx0 = jv + gbase
                gx1 = plsc.load_gather(gt_raw, [idx0])
                gy1 = plsc.load_gather(gt_raw, [idx0 + MP])
                gx2 = plsc.load_gather(gt_raw, [idx0 + 2 * MP])
                gy2 = plsc.load_gather(gt_raw, [idx0 + 3 * MP])
                ab = plsc.load_gather(gt_der, [idx0])
                nb, ni = [], []
                for v in range(nv):
                    ltx = jnp.maximum(ax1[v], gx1)
                    lty = jnp.maximum(ay1[v], gy1)
                    rbx = jnp.minimum(ax2[v], gx2)
                    rby = jnp.minimum(ay2[v], gy2)
                    wx = jnp.maximum(rbx - ltx, 0.0)
                    wy = jnp.maximum(rby - lty, 0.0)
                    inter = wx * wy
                    den = (aar[v] + ab) - inter + 1e-10
                    iou = inter / den
                    better = iou > bests[v]
                    nb.append(jnp.where(better, iou, bests[v]))
                    ni.append(jnp.where(better, jv, bestis[v]))
                return tuple(nb), tuple(ni)

            init = (tuple(jnp.full((L,), -1.0, jnp.float32) for _ in range(nv)),
                    tuple(jnp.zeros((L,), jnp.int32) for _ in range(nv)))
            bests, bestis = lax.fori_loop(0, M_GT, jbody, init)

            for v in range(nv):
                o = gb + v * L
                pos = bests[v] >= POS_IOU_THR
                pos_buf[bi, pl.ds(o, L)] = jnp.where(pos, ones, zeros)
                gi = bestis[v] + gbase
                gcx = plsc.load_gather(gt_der, [gi + MP])
                gcy = plsc.load_gather(gt_der, [gi + 2 * MP])
                lgw = plsc.load_gather(gt_der, [gi + 3 * MP])
                lgh = plsc.load_gather(gt_der, [gi + 4 * MP])
                clsf = plsc.load_gather(gt_der, [gi + 5 * MP])
                tcx = anc_der[1, pl.ds(o, L)]
                tcy = anc_der[2, pl.ds(o, L)]
                lw = anc_der[3, pl.ds(o, L)]
                lh = anc_der[4, pl.ds(o, L)]
                rows = iota + o
                reg_buf[bi, 0, pl.ds(o, L)] = jnp.where(pos, gcx - tcx, 0.0)
                reg_buf[bi, 1, pl.ds(o, L)] = jnp.where(pos, gcy - tcy, 0.0)
                reg_buf[bi, 2, pl.ds(o, L)] = jnp.where(pos, lgw - lw, 0.0)
                reg_buf[bi, 3, pl.ds(o, L)] = jnp.where(pos, lgh - lh, 0.0)
                reg_buf[bi, 4, pl.ds(o, L)] = zeros
                old = oc_buf[bi, pl.ds(o, L)]
                plsc.store_scatter(cls_buf.at[bi], [old, rows], zeros)
                cidx = clsf.astype(jnp.int32)
                plsc.store_scatter(cls_buf.at[bi], [cidx, rows], ones,
                                   mask=pos)
                oc_buf[bi, pl.ds(o, L)] = jnp.where(pos, cidx, 0)

        return run

    group4 = make_group(GROUP)
    group5 = make_group(5)

    def zero_chunk(i, _):
        o = i * L
        for p in range(2):
            oc_buf[p, pl.ds(o, L)] = jnp.zeros((L,), jnp.int32)
            for c in range(N_CLASSES):
                cls_buf[p, c, pl.ds(o, L)] = zeros
        return 0

    lax.fori_loop(0, NV, zero_chunk, 0)

    def out_copies(b, bi):
        return (
            pltpu.make_async_copy(
                pos_buf.at[bi], pos_hbm.at[b, pl.ds(base, TN)], sems.at[bi, 0]),
            pltpu.make_async_copy(
                cls_buf.at[bi], cls_hbm.at[:, b, pl.ds(base, TN)],
                sems.at[bi, 1]),
            pltpu.make_async_copy(
                reg_buf.at[bi], reg_hbm.at[:, b, pl.ds(base, TN)],
                sems.at[bi, 2]),
        )

    def batch_body(b, _):
        bi = b & 1

        @pl.when(b >= 2)
        def _():
            for c in out_copies(b - 2, bi):
                c.wait()

        def group_body(g, _):
            group4(b, bi, g * (GROUP * L))
            return 0

        lax.fori_loop(0, NGROUPS, group_body, 0)
        group5(b, bi, NGROUPS * GROUP * L)

        for c in out_copies(b, bi):
            c.start()
        return 0

    lax.fori_loop(0, BATCH, batch_body, 0)
    for b in (BATCH - 2, BATCH - 1):
        for c in out_copies(b, b & 1):
            c.wait()


@functools.partial(
    pl.kernel,
    out_type=(
        jax.ShapeDtypeStruct((BATCH, N_ANCHORS), jnp.float32),
        jax.ShapeDtypeStruct((N_CLASSES, BATCH, N_ANCHORS), jnp.float32),
        jax.ShapeDtypeStruct((5, BATCH, N_ANCHORS), jnp.float32),
    ),
    mesh=plsc.VectorSubcoreMesh(
        core_axis_name="c", subcore_axis_name="s",
        num_cores=NC, num_subcores=NS),
    compiler_params=pltpu.CompilerParams(
        needs_layout_passes=False, use_tc_tiling_on_sc=False),
    scratch_types=(
        pltpu.VMEM((4, TN), jnp.float32),
        pltpu.VMEM((5, TN), jnp.float32),
        pltpu.VMEM((BATCH * 6 * MP,), jnp.float32),
        pltpu.VMEM((BATCH * 6 * MP,), jnp.float32),
        pltpu.VMEM((2, TN), jnp.float32),
        pltpu.VMEM((2, N_CLASSES, TN), jnp.float32),
        pltpu.VMEM((2, 5, TN), jnp.float32),
        pltpu.VMEM((2, TN), jnp.int32),
        pltpu.SemaphoreType.DMA((2, 3)),
    ),
)
def _label_assign_sc(anc_hbm, gt_hbm, pos_hbm, cls_hbm, reg_hbm,
                     anc_raw, anc_der, gt_raw, gt_der,
                     pos_buf, cls_buf, reg_buf, oc_buf, sems):
    _sc_body(anc_hbm, gt_hbm, pos_hbm, cls_hbm, reg_hbm,
             anc_raw, anc_der, gt_raw, gt_der, pos_buf, cls_buf, reg_buf,
             oc_buf, sems)


def kernel(anchor, target, regressions, classifications):
    del regressions, classifications
    anc_t = anchor.T
    pad_row = jnp.array([0.0, 0.0, 1.0, 1.0, 0.0, 0.0], jnp.float32)
    tgt = jnp.concatenate(
        [target, jnp.broadcast_to(pad_row, (BATCH, MP - M_GT, 6))], axis=1)
    gt_t = jnp.transpose(tgt, (0, 2, 1)).reshape(-1)

    pos, cls, reg = _label_assign_sc(anc_t, gt_t)

    positive = pos > 0.5
    cls_out = jnp.transpose(cls, (1, 2, 0))
    reg_out = jnp.transpose(reg, (1, 2, 0))
    return positive, cls_out, reg_out
